# baseline (speedup 1.0000x reference)
import functools

import jax
import jax.numpy as jnp
from jax import lax
from jax.experimental import pallas as pl
from jax.experimental.pallas import tpu as pltpu
from jax.experimental.pallas import tpu_sc as plsc

_V = 1000000
_D = 32
_B = 16384
_NC = 2
_NS = 16
_L = 16
_NW = _NC * _NS
_BPW = _B // _NW
_CH = 128
_NQ = _BPW // _CH


def _glove_body(center2, outside2, cooc2, wt2,
                wc2, wo2, bc1, bo1, out_hbm,
                ridx_v, qidx_v, cidxf, oidxf,
                ceA, oeA, ceB, oeB, bc_v, bo_v, cw_v, wt_v, part_v,
                semA, semB, semC):
    wid = lax.axis_index("s") * _NC + lax.axis_index("c")

    pltpu.sync_copy(center2.at[wid], cidxf)
    pltpu.sync_copy(outside2.at[wid], oidxf)
    pltpu.sync_copy(cooc2.at[wid], cw_v)
    pltpu.sync_copy(wt2.at[wid], wt_v)

    for j in range(_BPW // _L):
        sl = pl.ds(j * _L, _L)
        ridx_v[sl] = lax.shift_right_logical(cidxf[sl], 2)
        qidx_v[sl] = lax.shift_right_logical(oidxf[sl], 2)

    bias_descs = []
    for j in range(_NQ):
        sl = pl.ds(j * _CH, _CH)
        bias_descs.append(
            pltpu.async_copy(bc1.at[cidxf.at[sl]], bc_v.at[sl], semC))
        bias_descs.append(
            pltpu.async_copy(bo1.at[oidxf.at[sl]], bo_v.at[sl], semC))

    def fire(q, ce, oe, sem):
        sl = pl.ds(q * _CH, _CH)
        return [pltpu.async_copy(wc2.at[ridx_v.at[sl]], ce, sem),
                pltpu.async_copy(wo2.at[qidx_v.at[sl]], oe, sem)]

    iota = lax.iota(jnp.int32, _L)

    def compute(q, ce, oe, tot):
        for b in range(_CH // _L):
            p0 = q * _CH + b * _L
            slot = b * _L + iota
            cv = cidxf[pl.ds(p0, _L)]
            ov = oidxf[pl.ds(p0, _L)]
            ccol = lax.bitwise_and(cv, 3) * _D
            ocol = lax.bitwise_and(ov, 3) * _D
            acc = jnp.zeros((_L,), jnp.float32)
            for d in range(_D):
                a = plsc.load_gather(ce, [slot, ccol + d])
                e = plsc.load_gather(oe, [slot, ocol + d])
                acc = acc + a * e
            err = acc + bc_v[pl.ds(p0, _L)] + bo_v[pl.ds(p0, _L)] \
                - cw_v[pl.ds(p0, _L)]
            tot = tot + wt_v[pl.ds(p0, _L)] * err * err
        return tot

    bufs = [(ceA, oeA, semA), (ceB, oeB, semB)]
    descs = {0: fire(0, *bufs[0])}
    tot = jnp.zeros((_L,), jnp.float32)
    for q in range(_NQ):
        if q + 1 < _NQ:
            descs[q + 1] = fire(q + 1, *bufs[(q + 1) % 2])
        for dsc in descs.pop(q):
            dsc.wait()
        if q == 0:
            for dsc in bias_descs:
                dsc.wait()
        ce, oe, _ = bufs[q % 2]
        tot = compute(q, ce, oe, tot)

    part_v[...] = tot
    pltpu.sync_copy(part_v, out_hbm.at[wid])


@jax.jit
def _glove(center2, outside2, cooc2, wt2, wc2, wo2, bc1, bo1):
    mesh = plsc.VectorSubcoreMesh(core_axis_name="c", subcore_axis_name="s")
    run = functools.partial(
        pl.kernel,
        mesh=mesh,
        compiler_params=pltpu.CompilerParams(
            needs_layout_passes=False, use_tc_tiling_on_sc=False),
        out_type=jax.ShapeDtypeStruct((_NW, _L), jnp.float32),
        scratch_types=[
            pltpu.VMEM((_BPW,), jnp.int32),
            pltpu.VMEM((_BPW,), jnp.int32),
            pltpu.VMEM((_BPW,), jnp.int32),
            pltpu.VMEM((_BPW,), jnp.int32),
            pltpu.VMEM((_CH, 128), jnp.float32),
            pltpu.VMEM((_CH, 128), jnp.float32),
            pltpu.VMEM((_CH, 128), jnp.float32),
            pltpu.VMEM((_CH, 128), jnp.float32),
            pltpu.VMEM((_BPW,), jnp.float32),
            pltpu.VMEM((_BPW,), jnp.float32),
            pltpu.VMEM((_BPW,), jnp.float32),
            pltpu.VMEM((_BPW,), jnp.float32),
            pltpu.VMEM((_L,), jnp.float32),
            pltpu.SemaphoreType.DMA,
            pltpu.SemaphoreType.DMA,
            pltpu.SemaphoreType.DMA,
        ],
    )(_glove_body)
    return run(center2, outside2, cooc2, wt2, wc2, wo2, bc1, bo1)


def kernel(center, outside, coocs, weighting, W_center, W_outside,
           b_center, b_outside):
    center2 = center.T.reshape(_NW, _BPW).astype(jnp.int32)
    outside2 = outside.T.reshape(_NW, _BPW).astype(jnp.int32)
    cooc2 = coocs.T.reshape(_NW, _BPW)
    wt2 = weighting.T.reshape(_NW, _BPW)
    wc2 = W_center.reshape(_V // 4, 128)
    wo2 = W_outside.reshape(_V // 4, 128)
    bc1 = b_center.T.reshape(_V)
    bo1 = b_outside.T.reshape(_V)
    partials = _glove(center2, outside2, cooc2, wt2, wc2, wo2, bc1, bo1)
    return jnp.sum(partials)
